# Initial kernel scaffold; baseline (speedup 1.0000x reference)
import jax
import jax.numpy as jnp
from jax.experimental import pallas as pl


def kernel(num_samples, temperature, logits, prior_logits, gumbel):
    raise NotImplementedError("write your pallas kernel here")


# speedup vs baseline: 1.2451x; 1.2451x over previous
import jax
import jax.numpy as jnp
from jax.experimental import pallas as pl
from jax.scipy.special import logsumexp

_ROWS = 256


def _gr_block(invt_ref, ln_ref, w_ref, c0_ref, g_ref, d_ref, s_ref):
    ln = ln_ref[...]
    w = w_ref[...]
    g = g_ref[...]
    R, K = g.shape

    z = (ln + g) * invt_ref[...]
    zmax = jnp.max(z, axis=-1, keepdims=True)
    e = jnp.exp(z - zmax)
    iota = jax.lax.broadcasted_iota(jnp.int32, (R, K), 1)
    idx = jnp.min(jnp.where(e >= 1.0, iota, K), axis=-1, keepdims=True)
    d_ref[...] = (iota == idx).astype(jnp.float32)

    eg = jnp.exp(-g)
    a = jnp.log(jnp.sum(eg, axis=-1, keepdims=True))
    b = jnp.log(jnp.sum(eg * w, axis=-1, keepdims=True))
    s_ref[...] = c0_ref[...] + K * (b - a)


def kernel(num_samples, temperature, logits, prior_logits, gumbel):
    K = logits.shape[-1]
    S = gumbel.shape[0]
    ln = (logits - logsumexp(logits, axis=0, keepdims=True)).reshape(1, K)
    w = jnp.exp(prior_logits.reshape(1, K) - ln)
    c0 = (jnp.sum(ln) - jnp.sum(prior_logits)).reshape(1, 1)
    invt = (1.0 / temperature).astype(jnp.float32).reshape(1, 1)

    grid = S // _ROWS
    D, s = pl.pallas_call(
        _gr_block,
        grid=(grid,),
        in_specs=[
            pl.BlockSpec((1, 1), lambda i: (0, 0)),
            pl.BlockSpec((1, K), lambda i: (0, 0)),
            pl.BlockSpec((1, K), lambda i: (0, 0)),
            pl.BlockSpec((1, 1), lambda i: (0, 0)),
            pl.BlockSpec((_ROWS, K), lambda i: (i, 0)),
        ],
        out_specs=[
            pl.BlockSpec((_ROWS, K), lambda i: (i, 0)),
            pl.BlockSpec((_ROWS, 1), lambda i: (i, 0)),
        ],
        out_shape=[
            jax.ShapeDtypeStruct((S, K), jnp.float32),
            jax.ShapeDtypeStruct((S, 1), jnp.float32),
        ],
    )(invt, ln, w, c0, gumbel)
    return (D, s.reshape(S))


# speedup vs baseline: 3.5161x; 2.8240x over previous
import jax
import jax.numpy as jnp
from jax.experimental import pallas as pl
from jax.scipy.special import logsumexp

_COLS = 512


def _gr_block(invt_ref, ln_ref, w_ref, c0_ref, g_ref, d_ref, s_ref):
    ln = ln_ref[...]
    w = w_ref[...]
    g = g_ref[...]
    K, C = g.shape

    z = (ln + g) * invt_ref[...]
    zmax = jnp.max(z, axis=0, keepdims=True)
    e = jnp.exp(z - zmax)
    iota = jax.lax.broadcasted_iota(jnp.int32, (K, C), 0)
    idx = jnp.min(jnp.where(e >= 1.0, iota, K), axis=0, keepdims=True)
    d_ref[...] = (iota == idx).astype(jnp.float32)

    eg = jnp.exp(-g)
    a = jnp.log(jnp.sum(eg, axis=0, keepdims=True))
    b = jnp.log(jnp.sum(eg * w, axis=0, keepdims=True))
    s_ref[...] = c0_ref[...] + K * (b - a)


def kernel(num_samples, temperature, logits, prior_logits, gumbel):
    K = logits.shape[-1]
    S = gumbel.shape[0]
    ln = (logits - logsumexp(logits, axis=0, keepdims=True)).reshape(K, 1)
    w = jnp.exp(prior_logits.reshape(K, 1) - ln)
    c0 = (jnp.sum(ln) - jnp.sum(prior_logits)).reshape(1, 1)
    invt = (1.0 / temperature).astype(jnp.float32).reshape(1, 1)

    gt = gumbel.T
    grid = S // _COLS
    Dt, s = pl.pallas_call(
        _gr_block,
        grid=(grid,),
        in_specs=[
            pl.BlockSpec((1, 1), lambda i: (0, 0)),
            pl.BlockSpec((K, 1), lambda i: (0, 0)),
            pl.BlockSpec((K, 1), lambda i: (0, 0)),
            pl.BlockSpec((1, 1), lambda i: (0, 0)),
            pl.BlockSpec((K, _COLS), lambda i: (0, i)),
        ],
        out_specs=[
            pl.BlockSpec((K, _COLS), lambda i: (0, i)),
            pl.BlockSpec((1, _COLS), lambda i: (0, i)),
        ],
        out_shape=[
            jax.ShapeDtypeStruct((K, S), jnp.float32),
            jax.ShapeDtypeStruct((1, S), jnp.float32),
        ],
    )(invt, ln, w, c0, gt)
    return (Dt.T, s.reshape(S))
